# baseline (speedup 1.0000x reference)
import jax
import jax.numpy as jnp
from jax import lax
from jax.experimental import pallas as pl
from jax.experimental.pallas import tpu as pltpu

NUM_CHARTS = 8
LATENT_DIM = 1024
RANK = 512
B = 2048
T = 512
NT = B // T


def _body(z_ref, s_ref, t_ref, we_hbm, wd_hbm, c_ref, d_ref, out_ref,
          we_v, wd_v, se_sem, sd_sem):
    i = pl.program_id(0)

    @pl.when(i == 0)
    def _():
        for e in range(NUM_CHARTS):
            pltpu.make_async_copy(we_hbm.at[e], we_v.at[e],
                                  se_sem.at[e]).start()
        for e in range(NUM_CHARTS):
            pltpu.make_async_copy(wd_hbm.at[e], wd_v.at[e],
                                  sd_sem.at[e]).start()

    zb = z_ref[...]
    sid = s_ref[...]
    tid = t_ref[...]
    lane8 = lax.broadcasted_iota(jnp.int32, (T, NUM_CHARTS), 1)
    oh_s = (sid == lane8)
    oh_t = (tid == lane8)

    h = jnp.zeros((T, RANK), jnp.float32)
    for e in range(NUM_CHARTS):
        @pl.when(i == 0)
        def _(e=e):
            pltpu.make_async_copy(we_hbm.at[e], we_v.at[e],
                                  se_sem.at[e]).wait()
        part = lax.dot_general(zb, we_v[e], (((1,), (1,)), ((), ())),
                               preferred_element_type=jnp.float32)
        h = jnp.where(oh_s[:, e:e + 1], part, h)
    h = h + lax.dot_general(oh_s.astype(jnp.float32), c_ref[...],
                            (((1,), (0,)), ((), ())),
                            preferred_element_type=jnp.float32)

    y = jnp.zeros((T, LATENT_DIM), jnp.float32)
    for e in range(NUM_CHARTS):
        @pl.when(i == 0)
        def _(e=e):
            pltpu.make_async_copy(wd_hbm.at[e], wd_v.at[e],
                                  sd_sem.at[e]).wait()
        part = lax.dot_general(h, wd_v[e], (((1,), (1,)), ((), ())),
                               preferred_element_type=jnp.float32)
        y = jnp.where(oh_t[:, e:e + 1], part, y)
    y = y + lax.dot_general(oh_t.astype(jnp.float32), d_ref[...],
                            (((1,), (0,)), ((), ())),
                            preferred_element_type=jnp.float32)
    out_ref[...] = y


@jax.jit
def kernel(z_n, source_idx, target_idx, W_enc, W_dec, c, d):
    src = source_idx.astype(jnp.int32).reshape(B, 1)
    tgt = target_idx.astype(jnp.int32).reshape(B, 1)
    return pl.pallas_call(
        _body,
        grid=(NT,),
        in_specs=[
            pl.BlockSpec((T, LATENT_DIM), lambda i: (i, 0)),
            pl.BlockSpec((T, 1), lambda i: (i, 0)),
            pl.BlockSpec((T, 1), lambda i: (i, 0)),
            pl.BlockSpec(memory_space=pl.ANY),
            pl.BlockSpec(memory_space=pl.ANY),
            pl.BlockSpec((NUM_CHARTS, RANK), lambda i: (0, 0)),
            pl.BlockSpec((NUM_CHARTS, LATENT_DIM), lambda i: (0, 0)),
        ],
        out_specs=pl.BlockSpec((T, LATENT_DIM), lambda i: (i, 0)),
        out_shape=jax.ShapeDtypeStruct((B, LATENT_DIM), jnp.float32),
        scratch_shapes=[
            pltpu.VMEM((NUM_CHARTS, RANK, LATENT_DIM), jnp.float32),
            pltpu.VMEM((NUM_CHARTS, LATENT_DIM, RANK), jnp.float32),
            pltpu.SemaphoreType.DMA((NUM_CHARTS,)),
            pltpu.SemaphoreType.DMA((NUM_CHARTS,)),
        ],
    )(z_n, src, tgt, W_enc, W_dec, c, d)
